# baseline (speedup 1.0000x reference)
import functools

import jax
import jax.numpy as jnp
from jax import lax
from jax.experimental import pallas as pl
from jax.experimental.pallas import tpu as pltpu
from jax.experimental.pallas import tpu_sc as plsc

VOCAB = 100000
HIDDEN = 768
MAX_POS = 2048
BATCH = 4
SEQ = 2048
EPS = 1e-12

NC = 2
NS = 16
NW = NC * NS
P_RANGE = SEQ // NW
NVR = HIDDEN // 16


def _vrsqrt(v):
    i = lax.bitcast_convert_type(v, jnp.int32)
    i = jnp.int32(0x5F3759DF) - (i >> 1)
    y = lax.bitcast_convert_type(i, jnp.float32)
    for _ in range(3):
        y = y * (1.5 - 0.5 * v * y * y)
    return y


def _sc_body(ids_hbm, tids_hbm, word_hbm, pos_hbm, type_hbm, g_hbm, b_hbm,
             out_hbm, ids_v, tids_v, pos_v, type_v, g_v, b_v, wbuf, sem):
    wid = lax.axis_index("s") * NC + lax.axis_index("c")
    pbase = wid * P_RANGE

    pltpu.sync_copy(pos_hbm.at[pl.ds(pbase, P_RANGE)], pos_v)
    pltpu.sync_copy(type_hbm, type_v)
    pltpu.sync_copy(g_hbm, g_v)
    pltpu.sync_copy(b_hbm, b_v)

    def one_token(i, t):
        def p1(j, carry):
            vsum, vsq = carry
            off = pl.ds(j * 16, 16)
            x = wbuf[i, off] + pos_v[i, off] + type_v[t, off]
            wbuf[i, off] = x
            return vsum + x, vsq + x * x

        zero = jnp.zeros((16,), jnp.float32)
        vsum, vsq = lax.fori_loop(0, NVR, p1, (zero, zero), unroll=4)
        s = lax.reduce_sum_p.bind(vsum, axes=(0,))
        sq = lax.reduce_sum_p.bind(vsq, axes=(0,))
        mean = s * (1.0 / HIDDEN)
        var = sq * (1.0 / HIDDEN) - mean * mean
        mean_v = jnp.full((16,), mean, jnp.float32)
        rstd_v = _vrsqrt(jnp.full((16,), var + EPS, jnp.float32))

        def p2(j, carry):
            off = pl.ds(j * 16, 16)
            y = (wbuf[i, off] - mean_v) * rstd_v
            wbuf[i, off] = y * g_v[off] + b_v[off]
            return carry

        lax.fori_loop(0, NVR, p2, 0, unroll=4)

    def batch_body(b, _):
        row0 = pl.multiple_of(b * SEQ + pbase, P_RANGE)
        pltpu.sync_copy(ids_hbm.at[pl.ds(row0, P_RANGE)], ids_v)
        pltpu.sync_copy(tids_hbm.at[pl.ds(row0, P_RANGE)], tids_v)
        pltpu.async_copy(word_hbm.at[ids_v], wbuf, sem).wait()

        def group(g, _):
            tvec = tids_v[pl.ds(g * 16, 16)]
            for k in range(16):
                one_token(g * 16 + k, tvec[k])
            return _

        lax.fori_loop(0, P_RANGE // 16, group, 0)
        pltpu.sync_copy(wbuf, out_hbm.at[pl.ds(row0, P_RANGE)])
        return _

    lax.fori_loop(0, BATCH, batch_body, 0)


@jax.jit
def _bert_embed_sc(ids_flat, tids_flat, word_emb, pos_emb, type_emb,
                   ln_gamma, ln_beta):
    mesh = plsc.VectorSubcoreMesh(core_axis_name="c", subcore_axis_name="s")
    run = pl.kernel(
        _sc_body,
        out_type=jax.ShapeDtypeStruct((BATCH * SEQ, HIDDEN), jnp.float32),
        mesh=mesh,
        compiler_params=pltpu.CompilerParams(needs_layout_passes=False),
        scratch_types=[
            pltpu.VMEM((P_RANGE,), jnp.int32),
            pltpu.VMEM((P_RANGE,), jnp.int32),
            pltpu.VMEM((P_RANGE, HIDDEN), jnp.float32),
            pltpu.VMEM((2, HIDDEN), jnp.float32),
            pltpu.VMEM((HIDDEN,), jnp.float32),
            pltpu.VMEM((HIDDEN,), jnp.float32),
            pltpu.VMEM((P_RANGE, HIDDEN), jnp.float32),
            pltpu.SemaphoreType.DMA,
        ],
    )
    return run(ids_flat, tids_flat, word_emb, pos_emb, type_emb,
               ln_gamma, ln_beta)


def kernel(input_ids, token_type_ids, word_emb, pos_emb, type_emb,
           ln_gamma, ln_beta):
    ids_flat = input_ids.reshape(-1).astype(jnp.int32)
    tids_flat = token_type_ids.reshape(-1).astype(jnp.int32)
    out = _bert_embed_sc(ids_flat, tids_flat, word_emb, pos_emb, type_emb,
                         ln_gamma, ln_beta)
    return out.reshape(BATCH, SEQ, HIDDEN)


# speedup vs baseline: 2.1436x; 2.1436x over previous
import jax
import jax.numpy as jnp
from jax import lax
from jax.experimental import pallas as pl
from jax.experimental.pallas import tpu as pltpu
from jax.experimental.pallas import tpu_sc as plsc

VOCAB = 100000
HIDDEN = 768
MAX_POS = 2048
BATCH = 4
SEQ = 2048
EPS = 1e-12

NC = 2
NS = 16
NW = NC * NS
P_RANGE = SEQ // NW
CHUNK = 32
NVR = HIDDEN // 16


def _vrsqrt(v):
    i = lax.bitcast_convert_type(v, jnp.int32)
    i = jnp.int32(0x5F3759DF) - (i >> 1)
    y = lax.bitcast_convert_type(i, jnp.float32)
    for _ in range(3):
        y = y * (1.5 - 0.5 * v * y * y)
    return y


def _sc_body(ids_hbm, tids_hbm, word_hbm, pos_hbm, type_hbm, g_hbm, b_hbm,
             out_hbm, ids_v, tids_v, pos_v, type_v, g_v, b_v, wbuf, xbuf,
             sem):
    wid = lax.axis_index("s") * NC + lax.axis_index("c")
    pbase = wid * P_RANGE

    pltpu.sync_copy(pos_hbm.at[pl.ds(pbase, P_RANGE)], pos_v)
    pltpu.sync_copy(type_hbm, type_v)
    pltpu.sync_copy(g_hbm, g_v)
    pltpu.sync_copy(b_hbm, b_v)

    def one_token(i, pi, t):
        @plsc.parallel_loop(0, NVR, unroll=4,
                            carry=(jnp.zeros((16,), jnp.float32),
                                   jnp.zeros((16,), jnp.float32)))
        def p1(j, carry):
            vsum, vsq = carry
            off = pl.ds(j * 16, 16)
            x = wbuf[i, off] + pos_v[pi, off] + type_v[t, off]
            xbuf[i, off] = x
            return vsum + x, vsq + x * x

        vsum, vsq = p1
        s = lax.reduce_sum_p.bind(vsum, axes=(0,))
        sq = lax.reduce_sum_p.bind(vsq, axes=(0,))
        mean = s * (1.0 / HIDDEN)
        var = sq * (1.0 / HIDDEN) - mean * mean
        mean_v = jnp.full((16,), mean, jnp.float32)
        rstd_v = _vrsqrt(jnp.full((16,), var + EPS, jnp.float32))

        @plsc.parallel_loop(0, NVR, unroll=4)
        def p2(j):
            off = pl.ds(j * 16, 16)
            y = (xbuf[i, off] - mean_v) * rstd_v
            wbuf[i, off] = y * g_v[off] + b_v[off]

    def chunk_body(c, _):
        row0 = pl.multiple_of(((c // 2) * SEQ) + pbase + (c % 2) * CHUNK,
                              CHUNK)
        prow0 = pl.multiple_of((c % 2) * CHUNK, CHUNK)
        pltpu.sync_copy(ids_hbm.at[pl.ds(row0, CHUNK)], ids_v)
        pltpu.sync_copy(tids_hbm.at[pl.ds(row0, CHUNK)], tids_v)
        pltpu.async_copy(word_hbm.at[ids_v], wbuf, sem).wait()

        def group(g, _):
            tvec = tids_v[pl.ds(g * 16, 16)]
            for k in range(16):
                one_token(g * 16 + k, prow0 + g * 16 + k, tvec[k])
            return _

        lax.fori_loop(0, CHUNK // 16, group, 0)
        pltpu.sync_copy(wbuf, out_hbm.at[pl.ds(row0, CHUNK)])
        return _

    lax.fori_loop(0, BATCH * (P_RANGE // CHUNK), chunk_body, 0)


@jax.jit
def _bert_embed_sc(ids_flat, tids_flat, word_emb, pos_emb, type_emb,
                   ln_gamma, ln_beta):
    mesh = plsc.VectorSubcoreMesh(core_axis_name="c", subcore_axis_name="s")
    run = pl.kernel(
        _sc_body,
        out_type=jax.ShapeDtypeStruct((BATCH * SEQ, HIDDEN), jnp.float32),
        mesh=mesh,
        compiler_params=pltpu.CompilerParams(needs_layout_passes=False),
        scratch_types=[
            pltpu.VMEM((CHUNK,), jnp.int32),
            pltpu.VMEM((CHUNK,), jnp.int32),
            pltpu.VMEM((P_RANGE, HIDDEN), jnp.float32),
            pltpu.VMEM((2, HIDDEN), jnp.float32),
            pltpu.VMEM((HIDDEN,), jnp.float32),
            pltpu.VMEM((HIDDEN,), jnp.float32),
            pltpu.VMEM((CHUNK, HIDDEN), jnp.float32),
            pltpu.VMEM((CHUNK, HIDDEN), jnp.float32),
            pltpu.SemaphoreType.DMA,
        ],
    )
    return run(ids_flat, tids_flat, word_emb, pos_emb, type_emb,
               ln_gamma, ln_beta)


def kernel(input_ids, token_type_ids, word_emb, pos_emb, type_emb,
           ln_gamma, ln_beta):
    ids_flat = input_ids.reshape(-1).astype(jnp.int32)
    tids_flat = token_type_ids.reshape(-1).astype(jnp.int32)
    out = _bert_embed_sc(ids_flat, tids_flat, word_emb, pos_emb, type_emb,
                         ln_gamma, ln_beta)
    return out.reshape(BATCH, SEQ, HIDDEN)


# speedup vs baseline: 2.2869x; 1.0669x over previous
import jax
import jax.numpy as jnp
from jax import lax
from jax.experimental import pallas as pl
from jax.experimental.pallas import tpu as pltpu
from jax.experimental.pallas import tpu_sc as plsc

VOCAB = 100000
HIDDEN = 768
MAX_POS = 2048
BATCH = 4
SEQ = 2048
EPS = 1e-12

NC = 2
NS = 16
NW = NC * NS
P_RANGE = SEQ // NW
CHUNK = 16
NCH = BATCH * (P_RANGE // CHUNK)
NVR = HIDDEN // 16
NBUF = 3


def _vrsqrt(v):
    i = lax.bitcast_convert_type(v, jnp.int32)
    i = jnp.int32(0x5F3759DF) - (i >> 1)
    y = lax.bitcast_convert_type(i, jnp.float32)
    for _ in range(3):
        y = y * (1.5 - 0.5 * v * y * y)
    return y


def _sc_body(ids_hbm, tids_hbm, word_hbm, pos_hbm, type_hbm, g_hbm, b_hbm,
             out_hbm, ids_v, tids_v, pos_v, type_v, g_v, b_v, buf, xbuf,
             gsem, osem):
    wid = lax.axis_index("s") * NC + lax.axis_index("c")
    pbase = wid * P_RANGE

    for b in range(BATCH):
        src = pl.ds(b * SEQ + pbase, P_RANGE)
        dst = pl.ds(b * P_RANGE, P_RANGE)
        pltpu.sync_copy(ids_hbm.at[src], ids_v.at[dst])
        pltpu.sync_copy(tids_hbm.at[src], tids_v.at[dst])
    pltpu.sync_copy(pos_hbm.at[pl.ds(pbase, P_RANGE)], pos_v)
    pltpu.sync_copy(type_hbm, type_v)
    pltpu.sync_copy(g_hbm, g_v)
    pltpu.sync_copy(b_hbm, b_v)

    def fire_gather(c):
        s = lax.rem(c, NBUF)
        pltpu.async_copy(word_hbm.at[ids_v.at[pl.ds(c * CHUNK, CHUNK)]],
                         buf.at[s], gsem.at[s])

    def wait_gather(c):
        s = lax.rem(c, NBUF)
        pltpu.make_async_copy(
            word_hbm.at[ids_v.at[pl.ds(c * CHUNK, CHUNK)]],
            buf.at[s], gsem.at[s]).wait()

    def wait_out(slot):
        pltpu.make_async_copy(buf.at[slot], out_hbm.at[pl.ds(0, CHUNK)],
                              osem.at[slot]).wait()

    def one_token(s, i, pi, t):
        @plsc.parallel_loop(0, NVR, unroll=6,
                            carry=(jnp.zeros((16,), jnp.float32),
                                   jnp.zeros((16,), jnp.float32)))
        def p1(j, carry):
            vsum, vsq = carry
            off = pl.ds(j * 16, 16)
            x = buf[s, i, off] + pos_v[pi, off] + type_v[t, off]
            xbuf[i, off] = x
            return vsum + x, vsq + x * x

        vsum, vsq = p1
        ssum = lax.reduce_sum_p.bind(vsum, axes=(0,))
        ssq = lax.reduce_sum_p.bind(vsq, axes=(0,))
        mean = ssum * (1.0 / HIDDEN)
        var = ssq * (1.0 / HIDDEN) - mean * mean
        mean_v = jnp.full((16,), mean, jnp.float32)
        rstd_v = _vrsqrt(jnp.full((16,), var + EPS, jnp.float32))

        @plsc.parallel_loop(0, NVR, unroll=6)
        def p2(j):
            off = pl.ds(j * 16, 16)
            y = (xbuf[i, off] - mean_v) * rstd_v
            buf[s, i, off] = y * g_v[off] + b_v[off]

    fire_gather(0)

    def chunk_body(c, _):
        s = lax.rem(c, NBUF)
        @pl.when(c < NCH - 1)
        def _prefetch():
            @pl.when(c >= 2)
            def _drain():
                wait_out(lax.rem(c + 1, NBUF))
            fire_gather(c + 1)

        wait_gather(c)
        prow0 = lax.rem(c, P_RANGE // CHUNK) * CHUNK
        tvec = tids_v[pl.ds(c * CHUNK, CHUNK)]
        for k in range(CHUNK):
            one_token(s, k, prow0 + k, tvec[k])

        row0 = (lax.div(c, P_RANGE // CHUNK) * SEQ + pbase
                + lax.rem(c, P_RANGE // CHUNK) * CHUNK)
        pltpu.async_copy(buf.at[s], out_hbm.at[pl.ds(row0, CHUNK)],
                         osem.at[s])
        return _

    lax.fori_loop(0, NCH, chunk_body, 0)
    for c in range(NCH - NBUF, NCH):
        wait_out(c % NBUF)


@jax.jit
def _bert_embed_sc(ids_flat, tids_flat, word_emb, pos_emb, type_emb,
                   ln_gamma, ln_beta):
    mesh = plsc.VectorSubcoreMesh(core_axis_name="c", subcore_axis_name="s")
    run = pl.kernel(
        _sc_body,
        out_type=jax.ShapeDtypeStruct((BATCH * SEQ, HIDDEN), jnp.float32),
        mesh=mesh,
        compiler_params=pltpu.CompilerParams(needs_layout_passes=False),
        scratch_types=[
            pltpu.VMEM((BATCH * P_RANGE,), jnp.int32),
            pltpu.VMEM((BATCH * P_RANGE,), jnp.int32),
            pltpu.VMEM((P_RANGE, HIDDEN), jnp.float32),
            pltpu.VMEM((2, HIDDEN), jnp.float32),
            pltpu.VMEM((HIDDEN,), jnp.float32),
            pltpu.VMEM((HIDDEN,), jnp.float32),
            pltpu.VMEM((NBUF, CHUNK, HIDDEN), jnp.float32),
            pltpu.VMEM((CHUNK, HIDDEN), jnp.float32),
            pltpu.SemaphoreType.DMA((NBUF,)),
            pltpu.SemaphoreType.DMA((NBUF,)),
        ],
    )
    return run(ids_flat, tids_flat, word_emb, pos_emb, type_emb,
               ln_gamma, ln_beta)


def kernel(input_ids, token_type_ids, word_emb, pos_emb, type_emb,
           ln_gamma, ln_beta):
    ids_flat = input_ids.reshape(-1).astype(jnp.int32)
    tids_flat = token_type_ids.reshape(-1).astype(jnp.int32)
    out = _bert_embed_sc(ids_flat, tids_flat, word_emb, pos_emb, type_emb,
                         ln_gamma, ln_beta)
    return out.reshape(BATCH, SEQ, HIDDEN)


# speedup vs baseline: 2.3878x; 1.0441x over previous
import jax
import jax.numpy as jnp
from jax import lax
from jax.experimental import pallas as pl
from jax.experimental.pallas import tpu as pltpu
from jax.experimental.pallas import tpu_sc as plsc

VOCAB = 100000
HIDDEN = 768
MAX_POS = 2048
BATCH = 4
SEQ = 2048
EPS = 1e-12

NC = 2
NS = 16
NW = NC * NS
P_RANGE = SEQ // NW
CHUNK = 16
NCH = BATCH * (P_RANGE // CHUNK)
NVR = HIDDEN // 16
NBUF = 3


def _vrsqrt(v):
    i = lax.bitcast_convert_type(v, jnp.int32)
    i = jnp.int32(0x5F3759DF) - (i >> 1)
    y = lax.bitcast_convert_type(i, jnp.float32)
    for _ in range(2):
        y = y * (1.5 - 0.5 * v * y * y)
    return y


def _sc_body(ids_hbm, tids_hbm, word_hbm, pos_hbm, type_hbm, g_hbm, b_hbm,
             out_hbm, ids_v, tids_v, pos_v, type_v, g_v, b_v, buf, xbuf,
             gsem, osem):
    wid = lax.axis_index("s") * NC + lax.axis_index("c")
    pbase = wid * P_RANGE

    for b in range(BATCH):
        src = pl.ds(b * SEQ + pbase, P_RANGE)
        dst = pl.ds(b * P_RANGE, P_RANGE)
        pltpu.sync_copy(ids_hbm.at[src], ids_v.at[dst])
        pltpu.sync_copy(tids_hbm.at[src], tids_v.at[dst])
    pltpu.sync_copy(pos_hbm.at[pl.ds(pbase, P_RANGE)], pos_v)
    pltpu.sync_copy(type_hbm, type_v)
    pltpu.sync_copy(g_hbm, g_v)
    pltpu.sync_copy(b_hbm, b_v)

    def fire_gather(c):
        s = lax.rem(c, NBUF)
        pltpu.async_copy(word_hbm.at[ids_v.at[pl.ds(c * CHUNK, CHUNK)]],
                         buf.at[s], gsem.at[s])

    def wait_gather(c):
        s = lax.rem(c, NBUF)
        pltpu.make_async_copy(
            word_hbm.at[ids_v.at[pl.ds(c * CHUNK, CHUNK)]],
            buf.at[s], gsem.at[s]).wait()

    def wait_out(slot):
        pltpu.make_async_copy(buf.at[slot], out_hbm.at[pl.ds(0, CHUNK)],
                              osem.at[slot]).wait()

    def one_token(s, i, pi, t):
        @plsc.parallel_loop(0, NVR, unroll=8,
                            carry=(jnp.zeros((16,), jnp.float32),
                                   jnp.zeros((16,), jnp.float32)))
        def p1(j, carry):
            vsum, vsq = carry
            off = pl.ds(j * 16, 16)
            x = buf[s, i, off] + pos_v[pi, off] + type_v[t, off]
            xbuf[i, off] = x
            return vsum + x, vsq + x * x

        vsum, vsq = p1
        ssum = lax.reduce_sum_p.bind(vsum, axes=(0,))
        ssq = lax.reduce_sum_p.bind(vsq, axes=(0,))
        mean = ssum * (1.0 / HIDDEN)
        var = ssq * (1.0 / HIDDEN) - mean * mean
        mean_v = jnp.full((16,), mean, jnp.float32)
        rstd_v = _vrsqrt(jnp.full((16,), var + EPS, jnp.float32))

        @plsc.parallel_loop(0, NVR, unroll=8)
        def p2(j):
            off = pl.ds(j * 16, 16)
            y = (xbuf[i, off] - mean_v) * rstd_v
            buf[s, i, off] = y * g_v[off] + b_v[off]

    fire_gather(0)

    def chunk_body(c, _):
        s = lax.rem(c, NBUF)
        @pl.when(c < NCH - 1)
        def _prefetch():
            @pl.when(c >= 2)
            def _drain():
                wait_out(lax.rem(c + 1, NBUF))
            fire_gather(c + 1)

        wait_gather(c)
        prow0 = lax.rem(c, P_RANGE // CHUNK) * CHUNK
        tvec = tids_v[pl.ds(c * CHUNK, CHUNK)]
        for k in range(CHUNK):
            one_token(s, k, prow0 + k, tvec[k])

        row0 = (lax.div(c, P_RANGE // CHUNK) * SEQ + pbase
                + lax.rem(c, P_RANGE // CHUNK) * CHUNK)
        pltpu.async_copy(buf.at[s], out_hbm.at[pl.ds(row0, CHUNK)],
                         osem.at[s])
        return _

    lax.fori_loop(0, NCH, chunk_body, 0)
    for c in range(NCH - NBUF, NCH):
        wait_out(c % NBUF)


@jax.jit
def _bert_embed_sc(ids_flat, tids_flat, word_emb, pos_emb, type_emb,
                   ln_gamma, ln_beta):
    mesh = plsc.VectorSubcoreMesh(core_axis_name="c", subcore_axis_name="s")
    run = pl.kernel(
        _sc_body,
        out_type=jax.ShapeDtypeStruct((BATCH * SEQ, HIDDEN), jnp.float32),
        mesh=mesh,
        compiler_params=pltpu.CompilerParams(needs_layout_passes=False),
        scratch_types=[
            pltpu.VMEM((BATCH * P_RANGE,), jnp.int32),
            pltpu.VMEM((BATCH * P_RANGE,), jnp.int32),
            pltpu.VMEM((P_RANGE, HIDDEN), jnp.float32),
            pltpu.VMEM((2, HIDDEN), jnp.float32),
            pltpu.VMEM((HIDDEN,), jnp.float32),
            pltpu.VMEM((HIDDEN,), jnp.float32),
            pltpu.VMEM((NBUF, CHUNK, HIDDEN), jnp.float32),
            pltpu.VMEM((CHUNK, HIDDEN), jnp.float32),
            pltpu.SemaphoreType.DMA((NBUF,)),
            pltpu.SemaphoreType.DMA((NBUF,)),
        ],
    )
    return run(ids_flat, tids_flat, word_emb, pos_emb, type_emb,
               ln_gamma, ln_beta)


def kernel(input_ids, token_type_ids, word_emb, pos_emb, type_emb,
           ln_gamma, ln_beta):
    ids_flat = input_ids.reshape(-1).astype(jnp.int32)
    tids_flat = token_type_ids.reshape(-1).astype(jnp.int32)
    out = _bert_embed_sc(ids_flat, tids_flat, word_emb, pos_emb, type_emb,
                         ln_gamma, ln_beta)
    return out.reshape(BATCH, SEQ, HIDDEN)
